# baseline (speedup 1.0000x reference)
import functools

import jax
import jax.numpy as jnp
from jax import lax
from jax.experimental import pallas as pl
from jax.experimental.pallas import tpu as pltpu
from jax.experimental.pallas import tpu_sc as plsc

_FP = 32


def _fm_sc(B, F, V8, D):
    info = plsc.get_sparse_core_info()
    NC, NS, L = info.num_cores, info.num_subcores, info.num_lanes
    NW = NC * NS
    assert D == L and B % NW == 0
    b_per_w = B // NW
    C = 16
    n_chunks = b_per_w // C
    CF = C * F
    FP = _FP

    mesh = plsc.VectorSubcoreMesh(core_axis_name="c", subcore_axis_name="s")

    @functools.partial(
        pl.kernel,
        mesh=mesh,
        out_type=jax.ShapeDtypeStruct((B,), jnp.float32),
        compiler_params=pltpu.CompilerParams(needs_layout_passes=False),
        scratch_types=[
            pltpu.VMEM((CF,), jnp.int32),
            pltpu.VMEM((CF, 128), jnp.float32),
            pltpu.VMEM((C * FP,), jnp.float32),
            pltpu.VMEM((C * FP,), jnp.int32),
            pltpu.VMEM((C, D), jnp.float32),
            pltpu.VMEM((C,), jnp.float32),
            pltpu.SemaphoreType.DMA,
        ],
    )
    def fm(table_hbm, idxh_hbm, idxlo_hbm, vals_hbm, out_hbm,
           idxh_v, rows_v, vals_v, idxlo_v, diffs_v, out_v, sem):
        wid = lax.axis_index("s") * NC + lax.axis_index("c")
        lane = lax.iota(jnp.int32, L)

        def chunk_body(j, carry):
            base_s = wid * b_per_w + j * C
            pltpu.sync_copy(idxh_hbm.at[pl.ds(base_s * F, CF)], idxh_v)
            pltpu.sync_copy(idxlo_hbm.at[pl.ds(base_s * FP, C * FP)], idxlo_v)
            pltpu.sync_copy(vals_hbm.at[pl.ds(base_s * FP, C * FP)], vals_v)
            pltpu.async_copy(table_hbm.at[idxh_v], rows_v, sem).wait()

            def sample_body(b, carry2):
                p0 = b * F
                v0 = vals_v[pl.ds(b * FP, L)]
                v1 = vals_v[pl.ds(b * FP + L, L)]
                o0 = idxlo_v[pl.ds(b * FP, L)]
                o1 = idxlo_v[pl.ds(b * FP + L, L)]
                acc = jnp.zeros((L,), jnp.float32)
                acc2 = jnp.zeros((L,), jnp.float32)
                for f in range(F):
                    vf = v0[f] if f < L else v1[f - L]
                    of = o0[f] if f < L else o1[f - L]
                    row = rows_v[p0 + f, pl.ds(of, L)]
                    w = row * vf
                    acc = acc + w
                    acc2 = acc2 + w * w
                diffs_v[b, :] = acc * acc - acc2
                return carry2

            lax.fori_loop(0, C, sample_body, 0)

            def group_body(g, carry2):
                row = g * L + lane
                tot = jnp.zeros((L,), jnp.float32)
                for d in range(D):
                    col = jnp.full((L,), d, jnp.int32)
                    tot = tot + plsc.load_gather(diffs_v, [row, col])
                out_v[pl.ds(g * L, L)] = 0.5 * tot
                return carry2

            lax.fori_loop(0, C // L, group_body, 0)
            pltpu.sync_copy(out_v, out_hbm.at[pl.ds(base_s, C)])
            return carry

        lax.fori_loop(0, n_chunks, chunk_body, 0)

    return fm


def kernel(feature_indices, feature_values, embedding_weight):
    B, F = feature_indices.shape
    V, D = embedding_weight.shape
    table128 = embedding_weight.reshape(V // 8, 8 * D)
    idx_flat = feature_indices.reshape(B * F).astype(jnp.int32)
    idx_hi = idx_flat >> 3
    idx_lo = jnp.pad(
        (feature_indices.astype(jnp.int32) & 7) << 4, ((0, 0), (0, _FP - F))
    ).reshape(B * _FP)
    vals_pad = jnp.pad(feature_values, ((0, 0), (0, _FP - F))).reshape(B * _FP)
    out = _fm_sc(B, F, V // 8, D)(table128, idx_hi, idx_lo, vals_pad)
    return out.reshape(B, 1)
